# baseline (speedup 1.0000x reference)
import functools

import jax
import jax.numpy as jnp
from jax import lax
from jax.experimental import pallas as pl
from jax.experimental.pallas import tpu as pltpu
from jax.experimental.pallas import tpu_sc as plsc


def _make_sc_kernel(N, C, K, NC, NS, L):
    NW = NC * NS
    rows_per_w = N // NW
    R = 128
    NCHUNK = rows_per_w // R
    NRG = R // L
    U = 4

    mesh = plsc.VectorSubcoreMesh(core_axis_name="c", subcore_axis_name="s")

    @functools.partial(
        pl.kernel,
        mesh=mesh,
        out_type=jax.ShapeDtypeStruct((K, N), jnp.float32),
        compiler_params=pltpu.CompilerParams(needs_layout_passes=False),
        scratch_types=[
            pltpu.VMEM((R, C), jnp.float32),
            pltpu.VMEM((R, C), jnp.float32),
            pltpu.VMEM((K, R), jnp.float32),
            pltpu.VMEM((K, R), jnp.float32),
            pltpu.VMEM((K,), jnp.int32),
            pltpu.VMEM((K,), jnp.float32),
            pltpu.VMEM((K,), jnp.float32),
            pltpu.SemaphoreType.DMA,
            pltpu.SemaphoreType.DMA,
            pltpu.SemaphoreType.DMA,
            pltpu.SemaphoreType.DMA,
        ],
    )
    def sc_kernel(y_hbm, w_hbm, b_hbm, onn_hbm, out_hbm,
                  in0, in1, out0, out1, onn_v, w_v, b_v,
                  sem_in0, sem_in1, sem_out0, sem_out1):
        wid = lax.axis_index("s") * NC + lax.axis_index("c")
        pltpu.sync_copy(onn_hbm, onn_v)
        pltpu.sync_copy(w_hbm, w_v)
        pltpu.sync_copy(b_hbm, b_v)

        base = wid * rows_per_w
        inbufs = [in0, in1]
        outbufs = [out0, out1]
        sin = [sem_in0, sem_in1]
        sout = [sem_out0, sem_out1]

        in_copies = [
            pltpu.make_async_copy(
                y_hbm.at[pl.ds(base + c * R, R)],
                inbufs[c % 2], sin[c % 2])
            for c in range(NCHUNK)
        ]
        out_copies = [
            pltpu.make_async_copy(
                outbufs[c % 2],
                out_hbm.at[:, pl.ds(base + c * R, R)],
                sout[c % 2])
            for c in range(NCHUNK)
        ]

        iota = lax.iota(jnp.int32, L)
        rgvec = [iota + rg * L for rg in range(NRG)]

        in_copies[0].start()
        for c in range(NCHUNK):
            if c + 1 < NCHUNK:
                in_copies[c + 1].start()
            in_copies[c].wait()
            if c >= 2:
                out_copies[c - 2].wait()

            inbuf = inbufs[c % 2]
            outbuf = outbufs[c % 2]

            @plsc.parallel_loop(0, K, unroll=U)
            def body(k, inbuf=inbuf, outbuf=outbuf):
                kk = jnp.full((L,), k, dtype=jnp.int32)
                col = plsc.load_gather(onn_v, [kk])
                wk = plsc.load_gather(w_v, [kk])
                bk = plsc.load_gather(b_v, [kk])
                for rg in range(NRG):
                    v = plsc.load_gather(inbuf, [rgvec[rg], col])
                    outbuf[k, pl.ds(rg * L, L)] = v * wk + bk

            out_copies[c].start()

        out_copies[NCHUNK - 2].wait()
        out_copies[NCHUNK - 1].wait()

    return sc_kernel


def kernel(Y_full, weights, bias, output_node_order):
    N, C = Y_full.shape
    K = output_node_order.shape[0]
    info = plsc.get_sparse_core_info()
    NC, NS, L = info.num_cores, info.num_subcores, info.num_lanes

    sc_kernel = _make_sc_kernel(N, C, K, NC, NS, L)
    out_t = sc_kernel(Y_full, weights, bias, output_node_order.astype(jnp.int32))
    return out_t.T
